# baseline (speedup 1.0000x reference)
import functools

import jax
import jax.numpy as jnp
from jax import lax
from jax.experimental import pallas as pl
from jax.experimental.pallas import tpu as pltpu
from jax.experimental.pallas import tpu_sc as plsc

N = 32768
F = 256
NBINS = 8
LANES = 16
NC = 2
NS = 16
NW = NC * NS
ROWS_PER_W = N // NW
CHUNK = 256
NCHUNK = ROWS_PER_W // CHUNK
TRASH = F * NBINS
ACC = 4096
RED = ACC // NS


def _sc_body(x_hbm, out_hbm, inbuf, acc, shared, redbuf, outbuf):
    cid = lax.axis_index("c")
    tid = lax.axis_index("s")
    wid = cid * NS + tid

    lane8 = lax.iota(jnp.int32, LANES) * 8
    zero16 = jnp.zeros((LANES,), jnp.float32)
    for v in range(ACC // LANES):
        acc[pl.ds(v * LANES, LANES)] = zero16

    base_elem = wid * (ROWS_PER_W * F)

    for c in range(NCHUNK):
        pltpu.sync_copy(
            x_hbm.at[pl.ds(base_elem + c * (CHUNK * F), CHUNK * F)], inbuf
        )

        def body(r, carry):
            off = r * F
            for j in range(F // LANES):
                xv = inbuf[pl.ds(off + j * LANES, LANES)]
                e1 = jnp.exp(-xv)
                p = 8.0 / (1.0 + e1)
                ki = (p + 0.5).astype(jnp.int32)
                kf = ki.astype(jnp.float32)
                ew = jnp.exp(125.0 * (kf - p))
                s = 1.0 / (1.0 + ew)
                oms = ew * s
                base = lane8 + (j * LANES * NBINS)
                i1 = jnp.where(ki < NBINS, base + ki, TRASH)
                i0 = jnp.where(ki > 0, base + ki - 1, TRASH)
                plsc.addupdate_scatter(acc, [i1], s)
                plsc.addupdate_scatter(acc, [i0], oms)
            return carry

        lax.fori_loop(0, CHUNK, body, 0)

    pltpu.sync_copy(acc, shared.at[pl.ds(tid * ACC, ACC)])
    plsc.subcore_barrier()

    col0 = tid * RED
    for i in range(NS):
        pltpu.sync_copy(
            shared.at[pl.ds(i * ACC + col0, RED)],
            redbuf.at[pl.ds(i * RED, RED)],
        )
    for v in range(RED // LANES):
        tot = redbuf[pl.ds(v * LANES, LANES)]
        for i in range(1, NS):
            tot = tot + redbuf[pl.ds(i * RED + v * LANES, LANES)]
        outbuf[pl.ds(v * LANES, LANES)] = tot
    pltpu.sync_copy(outbuf, out_hbm.at[pl.ds(cid * ACC + col0, RED)])


_sc_hist = functools.partial(
    pl.kernel,
    out_type=jax.ShapeDtypeStruct((NC * ACC,), jnp.float32),
    mesh=plsc.VectorSubcoreMesh(core_axis_name="c", subcore_axis_name="s"),
    scratch_types=[
        pltpu.VMEM((CHUNK * F,), jnp.float32),
        pltpu.VMEM((ACC,), jnp.float32),
        pltpu.VMEM_SHARED((NS * ACC,), jnp.float32),
        pltpu.VMEM((NS * RED,), jnp.float32),
        pltpu.VMEM((RED,), jnp.float32),
    ],
    compiler_params=pltpu.CompilerParams(needs_layout_passes=False),
)(_sc_body)


def _combine_body(p_ref, o_ref):
    a = p_ref[pl.ds(0, F * NBINS)]
    b = p_ref[pl.ds(ACC, F * NBINS)]
    o_ref[...] = (a + b) * (1.0 / N)


_combine = pl.pallas_call(
    _combine_body,
    out_shape=jax.ShapeDtypeStruct((F * NBINS,), jnp.float32),
)


@jax.jit
def kernel(input):
    partials = _sc_hist(input.reshape(-1))
    return _combine(partials)


# speedup vs baseline: 5.2731x; 5.2731x over previous
import functools

import jax
import jax.numpy as jnp
from jax import lax
from jax.experimental import pallas as pl
from jax.experimental.pallas import tpu as pltpu
from jax.experimental.pallas import tpu_sc as plsc

N = 32768
F = 256
NBINS = 8
LANES = 16
NC = 2
NS = 16
NW = NC * NS
ROWS_PER_W = N // NW
CHUNK = 128
NCHUNK = ROWS_PER_W // CHUNK
VEC_PER_CHUNK = CHUNK * F // LANES
ACC = 4096
RED = ACC // NS


def _sc_body(x_hbm, out_hbm, inbuf0, inbuf1, acc, shared, redbuf, outbuf,
             sem0, sem1):
    cid = lax.axis_index("c")
    tid = lax.axis_index("s")
    wid = cid * NS + tid

    lane10 = lax.iota(jnp.int32, LANES) * 10
    zero16 = jnp.zeros((LANES,), jnp.float32)
    for v in range(ACC // LANES):
        acc[pl.ds(v * LANES, LANES)] = zero16

    base_elem = wid * (ROWS_PER_W * F)
    bufs = (inbuf0, inbuf1)
    sems = (sem0, sem1)

    def start(c):
        return pltpu.async_copy(
            x_hbm.at[pl.ds(base_elem + c * (CHUNK * F), CHUNK * F)],
            bufs[c % 2],
            sems[c % 2],
        )

    copies = {0: start(0)}
    for c in range(NCHUNK):
        if c + 1 < NCHUNK:
            copies[c + 1] = start(c + 1)
        copies.pop(c).wait()
        inbuf = bufs[c % 2]

        @plsc.parallel_loop(0, VEC_PER_CHUNK, unroll=8)
        def body(i):
            xv = inbuf[pl.ds(i * LANES, LANES)]
            jmod = lax.rem(i, F // LANES)
            base = lane10 + jmod * (LANES * 10)
            e1 = jnp.exp(-xv)
            p = 8.0 / (1.0 + e1)
            ki = (p + 0.5).astype(jnp.int32)
            kf = ki.astype(jnp.float32)
            ew = jnp.exp(125.0 * (kf - p))
            s = 1.0 / (1.0 + ew)
            oms = ew * s
            i0 = base + ki
            i1 = i0 + 1
            plsc.addupdate_scatter(acc, [i1], s)
            plsc.addupdate_scatter(acc, [i0], oms)

    pltpu.sync_copy(acc, shared.at[pl.ds(tid * ACC, ACC)])
    plsc.subcore_barrier()

    col0 = tid * RED
    for i in range(NS):
        pltpu.sync_copy(
            shared.at[pl.ds(i * ACC + col0, RED)],
            redbuf.at[pl.ds(i * RED, RED)],
        )
    for v in range(RED // LANES):
        tot = redbuf[pl.ds(v * LANES, LANES)]
        for i in range(1, NS):
            tot = tot + redbuf[pl.ds(i * RED + v * LANES, LANES)]
        outbuf[pl.ds(v * LANES, LANES)] = tot
    pltpu.sync_copy(outbuf, out_hbm.at[pl.ds(cid * ACC + col0, RED)])


_sc_hist = functools.partial(
    pl.kernel,
    out_type=jax.ShapeDtypeStruct((NC * ACC,), jnp.float32),
    mesh=plsc.VectorSubcoreMesh(core_axis_name="c", subcore_axis_name="s"),
    scratch_types=[
        pltpu.VMEM((CHUNK * F,), jnp.float32),
        pltpu.VMEM((CHUNK * F,), jnp.float32),
        pltpu.VMEM((ACC,), jnp.float32),
        pltpu.VMEM_SHARED((NS * ACC,), jnp.float32),
        pltpu.VMEM((NS * RED,), jnp.float32),
        pltpu.VMEM((RED,), jnp.float32),
        pltpu.SemaphoreType.DMA,
        pltpu.SemaphoreType.DMA,
    ],
    compiler_params=pltpu.CompilerParams(needs_layout_passes=False),
)(_sc_body)


def _combine_body(p_ref, o_ref):
    a = p_ref[pl.ds(0, F * 10)]
    b = p_ref[pl.ds(ACC, F * 10)]
    o_ref[...] = (a + b) * (1.0 / N)


_combine = pl.pallas_call(
    _combine_body,
    out_shape=jax.ShapeDtypeStruct((F * 10,), jnp.float32),
)


@jax.jit
def kernel(input):
    partials = _sc_hist(input.reshape(-1))
    hist10 = _combine(partials)
    return hist10.reshape(F, 10)[:, 1:9].reshape(-1)


# speedup vs baseline: 5.2732x; 1.0000x over previous
import functools

import jax
import jax.numpy as jnp
from jax import lax
from jax.experimental import pallas as pl
from jax.experimental.pallas import tpu as pltpu
from jax.experimental.pallas import tpu_sc as plsc

N = 32768
F = 256
NBINS = 8
LANES = 16
NC = 2
NS = 16
NW = NC * NS
ROWS_PER_W = N // NW
CHUNK = 128
NCHUNK = ROWS_PER_W // CHUNK
VEC_PER_CHUNK = CHUNK * F // LANES
ACC = 4096
RED = ACC // NS


def _sc_body(x_hbm, out_hbm, inbuf0, inbuf1, acc, shared, redbuf, outbuf,
             sem0, sem1):
    cid = lax.axis_index("c")
    tid = lax.axis_index("s")
    wid = cid * NS + tid

    lane10 = lax.iota(jnp.int32, LANES) * 10
    zero16 = jnp.zeros((LANES,), jnp.float32)
    for v in range(ACC // LANES):
        acc[pl.ds(v * LANES, LANES)] = zero16

    base_elem = wid * (ROWS_PER_W * F)
    bufs = (inbuf0, inbuf1)
    sems = (sem0, sem1)

    def start(c):
        return pltpu.async_copy(
            x_hbm.at[pl.ds(base_elem + c * (CHUNK * F), CHUNK * F)],
            bufs[c % 2],
            sems[c % 2],
        )

    copies = {0: start(0)}
    for c in range(NCHUNK):
        if c + 1 < NCHUNK:
            copies[c + 1] = start(c + 1)
        copies.pop(c).wait()
        inbuf = bufs[c % 2]

        @plsc.parallel_loop(0, VEC_PER_CHUNK, unroll=8)
        def body(i):
            xv = inbuf[pl.ds(i * LANES, LANES)]
            jmod = lax.bitwise_and(i, (F // LANES) - 1)
            base = lane10 + jmod * (LANES * 10)
            e1 = jnp.exp(-xv)
            p = 8.0 / (1.0 + e1)
            ki = (p + 0.5).astype(jnp.int32)
            kf = ki.astype(jnp.float32)
            ew = jnp.exp(125.0 * (kf - p))
            s = 1.0 / (1.0 + ew)
            oms = ew * s
            i0 = base + ki
            i1 = i0 + 1
            plsc.addupdate_scatter(acc, [i1], s)
            plsc.addupdate_scatter(acc, [i0], oms)

    pltpu.sync_copy(acc, shared.at[pl.ds(tid * ACC, ACC)])
    plsc.subcore_barrier()

    col0 = tid * RED
    for i in range(NS):
        pltpu.sync_copy(
            shared.at[pl.ds(i * ACC + col0, RED)],
            redbuf.at[pl.ds(i * RED, RED)],
        )
    for v in range(RED // LANES):
        tot = redbuf[pl.ds(v * LANES, LANES)]
        for i in range(1, NS):
            tot = tot + redbuf[pl.ds(i * RED + v * LANES, LANES)]
        outbuf[pl.ds(v * LANES, LANES)] = tot
    pltpu.sync_copy(outbuf, out_hbm.at[pl.ds(cid * ACC + col0, RED)])


_sc_hist = functools.partial(
    pl.kernel,
    out_type=jax.ShapeDtypeStruct((NC * ACC,), jnp.float32),
    mesh=plsc.VectorSubcoreMesh(core_axis_name="c", subcore_axis_name="s"),
    scratch_types=[
        pltpu.VMEM((CHUNK * F,), jnp.float32),
        pltpu.VMEM((CHUNK * F,), jnp.float32),
        pltpu.VMEM((ACC,), jnp.float32),
        pltpu.VMEM_SHARED((NS * ACC,), jnp.float32),
        pltpu.VMEM((NS * RED,), jnp.float32),
        pltpu.VMEM((RED,), jnp.float32),
        pltpu.SemaphoreType.DMA,
        pltpu.SemaphoreType.DMA,
    ],
    compiler_params=pltpu.CompilerParams(needs_layout_passes=False),
)(_sc_body)


def _combine_body(p_ref, o_ref):
    a = p_ref[pl.ds(0, F * 10)]
    b = p_ref[pl.ds(ACC, F * 10)]
    o_ref[...] = (a + b) * (1.0 / N)


_combine = pl.pallas_call(
    _combine_body,
    out_shape=jax.ShapeDtypeStruct((F * 10,), jnp.float32),
)


@jax.jit
def kernel(input):
    partials = _sc_hist(input.reshape(-1))
    hist10 = _combine(partials)
    return hist10.reshape(F, 10)[:, 1:9].reshape(-1)


# speedup vs baseline: 7.3802x; 1.3996x over previous
import functools

import jax
import jax.numpy as jnp
from jax import lax
from jax.experimental import pallas as pl
from jax.experimental.pallas import tpu as pltpu
from jax.experimental.pallas import tpu_sc as plsc

N = 32768
F = 256
NBINS = 8
LANES = 16
NC = 2
NS = 16
NW = NC * NS
ROWS_SC = 16384
ROWS_TC = N - ROWS_SC
BR = 512
ROWS_PER_W = ROWS_SC // NW
CHUNK = 128
NCHUNK = ROWS_PER_W // CHUNK
VEC_PER_CHUNK = CHUNK * F // LANES
ACC = 4096
RED = ACC // NS


def _sc_body(x_hbm, out_hbm, inbuf0, inbuf1, acc, shared, redbuf, outbuf,
             sem0, sem1):
    cid = lax.axis_index("c")
    tid = lax.axis_index("s")
    wid = cid * NS + tid

    lane10 = lax.iota(jnp.int32, LANES) * 10
    zero16 = jnp.zeros((LANES,), jnp.float32)
    for v in range(ACC // LANES):
        acc[pl.ds(v * LANES, LANES)] = zero16

    base_elem = wid * (ROWS_PER_W * F)
    bufs = (inbuf0, inbuf1)
    sems = (sem0, sem1)

    def start(c):
        return pltpu.async_copy(
            x_hbm.at[pl.ds(base_elem + c * (CHUNK * F), CHUNK * F)],
            bufs[c % 2],
            sems[c % 2],
        )

    copies = {0: start(0)}
    for c in range(NCHUNK):
        if c + 1 < NCHUNK:
            copies[c + 1] = start(c + 1)
        copies.pop(c).wait()
        inbuf = bufs[c % 2]

        @plsc.parallel_loop(0, VEC_PER_CHUNK, unroll=8)
        def body(i):
            xv = inbuf[pl.ds(i * LANES, LANES)]
            jmod = lax.bitwise_and(i, (F // LANES) - 1)
            base = lane10 + jmod * (LANES * 10)
            e1 = jnp.exp(-xv)
            p = 8.0 / (1.0 + e1)
            ki = (p + 0.5).astype(jnp.int32)
            kf = ki.astype(jnp.float32)
            ew = jnp.exp(125.0 * (kf - p))
            s = 1.0 / (1.0 + ew)
            oms = ew * s
            i0 = base + ki
            i1 = i0 + 1
            plsc.addupdate_scatter(acc, [i1], s)
            plsc.addupdate_scatter(acc, [i0], oms)

    pltpu.sync_copy(acc, shared.at[pl.ds(tid * ACC, ACC)])
    plsc.subcore_barrier()

    col0 = tid * RED
    for i in range(NS):
        pltpu.sync_copy(
            shared.at[pl.ds(i * ACC + col0, RED)],
            redbuf.at[pl.ds(i * RED, RED)],
        )
    for v in range(RED // LANES):
        tot = redbuf[pl.ds(v * LANES, LANES)]
        for i in range(1, NS):
            tot = tot + redbuf[pl.ds(i * RED + v * LANES, LANES)]
        outbuf[pl.ds(v * LANES, LANES)] = tot
    pltpu.sync_copy(outbuf, out_hbm.at[pl.ds(cid * ACC + col0, RED)])


_sc_hist = functools.partial(
    pl.kernel,
    out_type=jax.ShapeDtypeStruct((NC * ACC,), jnp.float32),
    mesh=plsc.VectorSubcoreMesh(core_axis_name="c", subcore_axis_name="s"),
    scratch_types=[
        pltpu.VMEM((CHUNK * F,), jnp.float32),
        pltpu.VMEM((CHUNK * F,), jnp.float32),
        pltpu.VMEM((ACC,), jnp.float32),
        pltpu.VMEM_SHARED((NS * ACC,), jnp.float32),
        pltpu.VMEM((NS * RED,), jnp.float32),
        pltpu.VMEM((RED,), jnp.float32),
        pltpu.SemaphoreType.DMA,
        pltpu.SemaphoreType.DMA,
    ],
    compiler_params=pltpu.CompilerParams(needs_layout_passes=False),
)(_sc_body)


def _tc_body(x_ref, d_ref):
    x = x_ref[...]
    qh = 250.0 * jnp.tanh(x * 0.5) + 250.0
    prev = jnp.tanh(qh)
    cols = []
    for k in range(1, NBINS + 1):
        cur = jnp.tanh(qh - 62.5 * k)
        cols.append(0.5 * jnp.sum(prev - cur, axis=0))
        prev = cur
    d = jnp.stack(cols, axis=1)

    @pl.when(pl.program_id(0) == 0)
    def _():
        d_ref[...] = jnp.zeros_like(d_ref)

    d_ref[...] += d


_tc_edges = pl.pallas_call(
    _tc_body,
    grid=(ROWS_TC // BR,),
    in_specs=[pl.BlockSpec((BR, F), lambda i: (i + ROWS_SC // BR, 0))],
    out_specs=pl.BlockSpec((F, NBINS), lambda i: (0, 0)),
    out_shape=jax.ShapeDtypeStruct((F, NBINS), jnp.float32),
)


def _combine_body(p_ref, d_ref, o_ref):
    sc10 = p_ref[0] + p_ref[1]
    o_ref[...] = (sc10[:, 1:9] + d_ref[...]) * (1.0 / N)


_combine = pl.pallas_call(
    _combine_body,
    out_shape=jax.ShapeDtypeStruct((F, NBINS), jnp.float32),
)


@jax.jit
def kernel(input):
    partials = _sc_hist(input.reshape(-1))
    d = _tc_edges(input)
    a = partials.reshape(NC, ACC)[:, : F * 10].reshape(NC, F, 10)
    return _combine(a, d).reshape(-1)


# speedup vs baseline: 10.4574x; 1.4170x over previous
import functools

import jax
import jax.numpy as jnp
from jax import lax
from jax.experimental import pallas as pl
from jax.experimental.pallas import tpu as pltpu
from jax.experimental.pallas import tpu_sc as plsc

N = 32768
F = 256
NBINS = 8
LANES = 16
NC = 2
NS = 16
NW = NC * NS
ROWS_SC = 16384
ROWS_TC = N - ROWS_SC
BR = 512
ROWS_PER_W = ROWS_SC // NW
CHUNK = 128
NCHUNK = ROWS_PER_W // CHUNK
VEC_PER_CHUNK = CHUNK * F // LANES
ACC = 4096
RED = ACC // NS


def _sc_body(x_hbm, out_hbm, inbuf0, inbuf1, acc, shared, redbuf, outbuf,
             sem0, sem1):
    cid = lax.axis_index("c")
    tid = lax.axis_index("s")
    wid = cid * NS + tid

    lane10 = lax.iota(jnp.int32, LANES) * 10
    zero16 = jnp.zeros((LANES,), jnp.float32)
    for v in range(ACC // LANES):
        acc[pl.ds(v * LANES, LANES)] = zero16

    base_row = wid * ROWS_PER_W
    bufs = (inbuf0, inbuf1)
    sems = (sem0, sem1)

    def start(c):
        return pltpu.async_copy(
            x_hbm.at[pl.ds(base_row + c * CHUNK, CHUNK)],
            bufs[c % 2],
            sems[c % 2],
        )

    copies = {0: start(0)}
    for c in range(NCHUNK):
        if c + 1 < NCHUNK:
            copies[c + 1] = start(c + 1)
        copies.pop(c).wait()
        inbuf = bufs[c % 2]

        @plsc.parallel_loop(0, VEC_PER_CHUNK, unroll=8)
        def body(i):
            ri = lax.shift_right_logical(i, 4)
            jmod = lax.bitwise_and(i, (F // LANES) - 1)
            xv = inbuf[ri, pl.ds(jmod * LANES, LANES)]
            base = lane10 + jmod * (LANES * 10)
            e1 = jnp.exp(-xv)
            p = 8.0 / (1.0 + e1)
            ki = (p + 0.5).astype(jnp.int32)
            kf = ki.astype(jnp.float32)
            ew = jnp.exp(125.0 * (kf - p))
            s = 1.0 / (1.0 + ew)
            oms = ew * s
            i0 = base + ki
            i1 = i0 + 1
            plsc.addupdate_scatter(acc, [i1], s)
            plsc.addupdate_scatter(acc, [i0], oms)

    pltpu.sync_copy(acc, shared.at[pl.ds(tid * ACC, ACC)])
    plsc.subcore_barrier()

    col0 = tid * RED
    for i in range(NS):
        pltpu.sync_copy(
            shared.at[pl.ds(i * ACC + col0, RED)],
            redbuf.at[pl.ds(i * RED, RED)],
        )
    for v in range(RED // LANES):
        tot = redbuf[pl.ds(v * LANES, LANES)]
        for i in range(1, NS):
            tot = tot + redbuf[pl.ds(i * RED + v * LANES, LANES)]
        outbuf[pl.ds(v * LANES, LANES)] = tot
    pltpu.sync_copy(outbuf, out_hbm.at[pl.ds(cid * ACC + col0, RED)])


_sc_hist = functools.partial(
    pl.kernel,
    out_type=jax.ShapeDtypeStruct((NC * ACC,), jnp.float32),
    mesh=plsc.VectorSubcoreMesh(core_axis_name="c", subcore_axis_name="s"),
    scratch_types=[
        pltpu.VMEM((CHUNK, F), jnp.float32),
        pltpu.VMEM((CHUNK, F), jnp.float32),
        pltpu.VMEM((ACC,), jnp.float32),
        pltpu.VMEM_SHARED((NS * ACC,), jnp.float32),
        pltpu.VMEM((NS * RED,), jnp.float32),
        pltpu.VMEM((RED,), jnp.float32),
        pltpu.SemaphoreType.DMA,
        pltpu.SemaphoreType.DMA,
    ],
    compiler_params=pltpu.CompilerParams(needs_layout_passes=False),
)(_sc_body)


def _tc_body(x_ref, d_ref):
    x = x_ref[...]
    qh = 250.0 * jnp.tanh(x * 0.5) + 250.0
    prev = jnp.tanh(qh)
    cols = []
    for k in range(1, NBINS + 1):
        cur = jnp.tanh(qh - 62.5 * k)
        cols.append(0.5 * jnp.sum(prev - cur, axis=0))
        prev = cur
    d = jnp.stack(cols, axis=1)

    @pl.when(pl.program_id(0) == 0)
    def _():
        d_ref[...] = jnp.zeros_like(d_ref)

    d_ref[...] += d


_tc_edges = pl.pallas_call(
    _tc_body,
    grid=(ROWS_TC // BR,),
    in_specs=[pl.BlockSpec((BR, F), lambda i: (i + ROWS_SC // BR, 0))],
    out_specs=pl.BlockSpec((F, NBINS), lambda i: (0, 0)),
    out_shape=jax.ShapeDtypeStruct((F, NBINS), jnp.float32),
)


def _combine_body(p_ref, d_ref, o_ref):
    sc10 = p_ref[0] + p_ref[1]
    o_ref[...] = (sc10[:, 1:9] + d_ref[...]) * (1.0 / N)


_combine = pl.pallas_call(
    _combine_body,
    out_shape=jax.ShapeDtypeStruct((F, NBINS), jnp.float32),
)


@jax.jit
def kernel(input):
    partials = _sc_hist(input)
    d = _tc_edges(input)
    a = partials.reshape(NC, ACC)[:, : F * 10].reshape(NC, F, 10)
    return _combine(a, d).reshape(-1)


# speedup vs baseline: 12.1419x; 1.1611x over previous
import functools

import jax
import jax.numpy as jnp
from jax import lax
from jax.experimental import pallas as pl
from jax.experimental.pallas import tpu as pltpu
from jax.experimental.pallas import tpu_sc as plsc

N = 32768
F = 256
NBINS = 8
LANES = 16
NC = 2
NS = 16
NW = NC * NS
ROWS_SC = 12288
ROWS_TC = N - ROWS_SC
BR = 512
ROWS_PER_W = ROWS_SC // NW
CHUNK = 128
NCHUNK = ROWS_PER_W // CHUNK
VEC_PER_CHUNK = CHUNK * F // LANES
ACC = 4096
RED = ACC // NS


def _sc_body(x_hbm, out_hbm, inbuf0, inbuf1, acc, shared, redbuf, outbuf,
             sem0, sem1):
    cid = lax.axis_index("c")
    tid = lax.axis_index("s")
    wid = cid * NS + tid

    lane10 = lax.iota(jnp.int32, LANES) * 10
    zero16 = jnp.zeros((LANES,), jnp.float32)
    for v in range(ACC // LANES):
        acc[pl.ds(v * LANES, LANES)] = zero16

    base_row = wid * ROWS_PER_W
    bufs = (inbuf0, inbuf1)
    sems = (sem0, sem1)

    def start(c):
        return pltpu.async_copy(
            x_hbm.at[pl.ds(base_row + c * CHUNK, CHUNK)],
            bufs[c % 2],
            sems[c % 2],
        )

    copies = {0: start(0)}
    for c in range(NCHUNK):
        if c + 1 < NCHUNK:
            copies[c + 1] = start(c + 1)
        copies.pop(c).wait()
        inbuf = bufs[c % 2]

        @plsc.parallel_loop(0, VEC_PER_CHUNK, unroll=8)
        def body(i):
            ri = lax.shift_right_logical(i, 4)
            jmod = lax.bitwise_and(i, (F // LANES) - 1)
            xv = inbuf[ri, pl.ds(jmod * LANES, LANES)]
            base = lane10 + jmod * (LANES * 10)
            e1 = jnp.exp(-xv)
            p = 8.0 / (1.0 + e1)
            ki = (p + 0.5).astype(jnp.int32)
            kf = ki.astype(jnp.float32)
            ew = jnp.exp(125.0 * (kf - p))
            s = 1.0 / (1.0 + ew)
            oms = ew * s
            i0 = base + ki
            i1 = i0 + 1
            plsc.addupdate_scatter(acc, [i1], s)
            plsc.addupdate_scatter(acc, [i0], oms)

    pltpu.sync_copy(acc, shared.at[pl.ds(tid * ACC, ACC)])
    plsc.subcore_barrier()

    col0 = tid * RED
    for i in range(NS):
        pltpu.sync_copy(
            shared.at[pl.ds(i * ACC + col0, RED)],
            redbuf.at[pl.ds(i * RED, RED)],
        )
    for v in range(RED // LANES):
        tot = redbuf[pl.ds(v * LANES, LANES)]
        for i in range(1, NS):
            tot = tot + redbuf[pl.ds(i * RED + v * LANES, LANES)]
        outbuf[pl.ds(v * LANES, LANES)] = tot
    pltpu.sync_copy(outbuf, out_hbm.at[pl.ds(cid * ACC + col0, RED)])


_sc_hist = functools.partial(
    pl.kernel,
    out_type=jax.ShapeDtypeStruct((NC * ACC,), jnp.float32),
    mesh=plsc.VectorSubcoreMesh(core_axis_name="c", subcore_axis_name="s"),
    scratch_types=[
        pltpu.VMEM((CHUNK, F), jnp.float32),
        pltpu.VMEM((CHUNK, F), jnp.float32),
        pltpu.VMEM((ACC,), jnp.float32),
        pltpu.VMEM_SHARED((NS * ACC,), jnp.float32),
        pltpu.VMEM((NS * RED,), jnp.float32),
        pltpu.VMEM((RED,), jnp.float32),
        pltpu.SemaphoreType.DMA,
        pltpu.SemaphoreType.DMA,
    ],
    compiler_params=pltpu.CompilerParams(needs_layout_passes=False),
)(_sc_body)


def _tc_body(x_ref, d_ref):
    x = x_ref[...]
    qh = 250.0 * jnp.tanh(x * 0.5) + 250.0
    prev = jnp.tanh(qh)
    cols = []
    for k in range(1, NBINS + 1):
        cur = jnp.tanh(qh - 62.5 * k)
        cols.append(0.5 * jnp.sum(prev - cur, axis=0))
        prev = cur
    d = jnp.stack(cols, axis=1)

    @pl.when(pl.program_id(0) == 0)
    def _():
        d_ref[...] = jnp.zeros_like(d_ref)

    d_ref[...] += d


_tc_edges = pl.pallas_call(
    _tc_body,
    grid=(ROWS_TC // BR,),
    in_specs=[pl.BlockSpec((BR, F), lambda i: (i + ROWS_SC // BR, 0))],
    out_specs=pl.BlockSpec((F, NBINS), lambda i: (0, 0)),
    out_shape=jax.ShapeDtypeStruct((F, NBINS), jnp.float32),
)


def _combine_body(p_ref, d_ref, o_ref):
    sc10 = p_ref[0] + p_ref[1]
    o_ref[...] = (sc10[:, 1:9] + d_ref[...]) * (1.0 / N)


_combine = pl.pallas_call(
    _combine_body,
    out_shape=jax.ShapeDtypeStruct((F, NBINS), jnp.float32),
)


@jax.jit
def kernel(input):
    partials = _sc_hist(input)
    d = _tc_edges(input)
    a = partials.reshape(NC, ACC)[:, : F * 10].reshape(NC, F, 10)
    return _combine(a, d).reshape(-1)
